# speedup vs baseline: 1.2161x; 1.2161x over previous
import functools
import jax
import jax.numpy as jnp
from jax.experimental import pallas as pl
from jax.experimental.pallas import tpu as pltpu

_N = 10000
_K = 32
_H = 128
_BLOCK = 200


def _body(x_ref, mm_ref, mr_ref, w_ref, b_ref, m_ref, root_ref):
    inv_k = 1.0 / _K
    acc_m = jnp.sum(mm_ref[...], axis=1) * inv_k
    acc_r = jnp.sum(mr_ref[...], axis=1) * inv_k
    z = jnp.dot(x_ref[...], w_ref[...], preferred_element_type=jnp.float32)
    m_ref[...] = jnp.maximum(z + b_ref[...] + acc_m, 0.0)
    root_ref[...] = acc_r


def kernel(x, mailbox_m, mailbox_root, W_z, b_z):
    n = x.shape[0]
    grid = (n // _BLOCK,)
    wt = W_z.T
    b2 = b_z.reshape(1, _H)
    m, root = pl.pallas_call(
        _body,
        grid=grid,
        in_specs=[
            pl.BlockSpec((_BLOCK, _H), lambda i: (i, 0)),
            pl.BlockSpec((_BLOCK, _K, _H), lambda i: (i, 0, 0)),
            pl.BlockSpec((_BLOCK, _K, _H), lambda i: (i, 0, 0)),
            pl.BlockSpec((_H, _H), lambda i: (0, 0)),
            pl.BlockSpec((1, _H), lambda i: (0, 0)),
        ],
        out_specs=[
            pl.BlockSpec((_BLOCK, _H), lambda i: (i, 0)),
            pl.BlockSpec((_BLOCK, _H), lambda i: (i, 0)),
        ],
        out_shape=[
            jax.ShapeDtypeStruct((n, _H), jnp.float32),
            jax.ShapeDtypeStruct((n, _H), jnp.float32),
        ],
        compiler_params=pltpu.CompilerParams(
            dimension_semantics=("arbitrary",),
        ),
    )(x, mailbox_m, mailbox_root, wt, b2)
    return (m, root)
